# baseline (device time: 8686 ns/iter reference)
import jax
import jax.numpy as jnp
from jax import lax
from jax.experimental import pallas as pl
from jax.experimental.pallas import tpu as pltpu


def kernel(x):
    _, m, n2 = x.shape
    n = n2 // 2

    n_chunks = 2
    mc = m // n_chunks

    def body(x_ref, out_ref, send_buf, keep_buf, recv_buf,
             local_sems, out_sems, send_sems, recv_sems):
        my_x = lax.axis_index("x")
        my_y = lax.axis_index("y")
        my_z = lax.axis_index("z")
        partner = (1 - my_x, my_y, my_z)

        barrier_sem = pltpu.get_barrier_semaphore()
        pl.semaphore_signal(
            barrier_sem, inc=1,
            device_id=partner, device_id_type=pl.DeviceIdType.MESH,
        )

        off_send = (1 - my_x) * n
        off_keep = my_x * n

        cp_send = pltpu.make_async_copy(
            x_ref.at[0, :, pl.ds(off_send, n)], send_buf, local_sems.at[0]
        )
        cp_send.start()
        cp_keep = pltpu.make_async_copy(
            x_ref.at[0, :, pl.ds(off_keep, n)], keep_buf, local_sems.at[1]
        )
        cp_keep.start()

        cp_send.wait()

        rdmas = []
        for c in range(n_chunks):
            rows = pl.ds(c * mc, mc)
            rdma = pltpu.make_async_remote_copy(
                src_ref=send_buf.at[rows, :],
                dst_ref=recv_buf.at[rows, :],
                send_sem=send_sems.at[c],
                recv_sem=recv_sems.at[c],
                device_id=partner,
                device_id_type=pl.DeviceIdType.MESH,
            )
            rdma.start()
            rdmas.append(rdma)

        pl.semaphore_wait(barrier_sem, 1)
        cp_keep.wait()

        out_cps = []
        for c, rdma in enumerate(rdmas):
            rows = pl.ds(c * mc, mc)
            rdma.wait_recv()
            keep_buf[rows, :] = keep_buf[rows, :] + recv_buf[rows, :]
            out_cp = pltpu.make_async_copy(
                keep_buf.at[rows, :], out_ref.at[rows, :], out_sems.at[c]
            )
            out_cp.start()
            out_cps.append(out_cp)

        for out_cp in out_cps:
            out_cp.wait()
        for rdma in rdmas:
            rdma.wait_send()

    x = pltpu.with_memory_space_constraint(x, pltpu.MemorySpace.HBM)

    return pl.pallas_call(
        body,
        out_shape=jax.ShapeDtypeStruct((m, n), jnp.float32),
        in_specs=[pl.BlockSpec(memory_space=pl.ANY)],
        out_specs=pl.BlockSpec(memory_space=pl.ANY),
        scratch_shapes=[
            pltpu.VMEM((m, n), jnp.float32),
            pltpu.VMEM((m, n), jnp.float32),
            pltpu.VMEM((m, n), jnp.float32),
            pltpu.SemaphoreType.DMA((2,)),
            pltpu.SemaphoreType.DMA((n_chunks,)),
            pltpu.SemaphoreType.DMA((n_chunks,)),
            pltpu.SemaphoreType.DMA((n_chunks,)),
        ],
        compiler_params=pltpu.CompilerParams(collective_id=0),
    )(x)


# device time: 8481 ns/iter; 1.0242x vs baseline; 1.0242x over previous
import jax
import jax.numpy as jnp
from jax import lax
from jax.experimental import pallas as pl
from jax.experimental.pallas import tpu as pltpu


def kernel(x):
    _, m, n2 = x.shape
    n = n2 // 2

    def body(x_ref, out_ref, send_buf, keep_buf, recv_buf,
             local_sems, send_sem, recv_sem):
        my_x = lax.axis_index("x")
        my_y = lax.axis_index("y")
        my_z = lax.axis_index("z")
        partner = (1 - my_x, my_y, my_z)

        barrier_sem = pltpu.get_barrier_semaphore()
        pl.semaphore_signal(
            barrier_sem, inc=1,
            device_id=partner, device_id_type=pl.DeviceIdType.MESH,
        )

        off_send = (1 - my_x) * n
        off_keep = my_x * n

        cp_send = pltpu.make_async_copy(
            x_ref.at[0, :, pl.ds(off_send, n)], send_buf, local_sems.at[0]
        )
        cp_send.start()
        cp_keep = pltpu.make_async_copy(
            x_ref.at[0, :, pl.ds(off_keep, n)], keep_buf, local_sems.at[1]
        )
        cp_keep.start()

        cp_send.wait()

        rdma = pltpu.make_async_remote_copy(
            src_ref=send_buf,
            dst_ref=recv_buf,
            send_sem=send_sem,
            recv_sem=recv_sem,
            device_id=partner,
            device_id_type=pl.DeviceIdType.MESH,
        )
        rdma.start()
        pl.semaphore_wait(barrier_sem, 1)
        rdma.wait_recv()
        cp_keep.wait()

        out_ref[...] = keep_buf[...] + recv_buf[...]
        rdma.wait_send()

    x = pltpu.with_memory_space_constraint(x, pltpu.MemorySpace.HBM)

    return pl.pallas_call(
        body,
        out_shape=jax.ShapeDtypeStruct((m, n), jnp.float32),
        in_specs=[pl.BlockSpec(memory_space=pl.ANY)],
        out_specs=pl.BlockSpec(memory_space=pltpu.VMEM),
        scratch_shapes=[
            pltpu.VMEM((m, n), jnp.float32),
            pltpu.VMEM((m, n), jnp.float32),
            pltpu.VMEM((m, n), jnp.float32),
            pltpu.SemaphoreType.DMA((2,)),
            pltpu.SemaphoreType.DMA,
            pltpu.SemaphoreType.DMA,
        ],
        compiler_params=pltpu.CompilerParams(collective_id=0),
    )(x)


# device time: 8444 ns/iter; 1.0287x vs baseline; 1.0044x over previous
import jax
import jax.numpy as jnp
from jax import lax
from jax.experimental import pallas as pl
from jax.experimental.pallas import tpu as pltpu


def kernel(x):
    _, m, n2 = x.shape
    n = n2 // 2

    n_chunks = 2
    mc = m // n_chunks

    def body(x_ref, out_ref, send_buf, keep_buf, recv_buf,
             local_sems, send_sems, recv_sems):
        my_x = lax.axis_index("x")
        my_y = lax.axis_index("y")
        my_z = lax.axis_index("z")
        partner = (1 - my_x, my_y, my_z)

        barrier_sem = pltpu.get_barrier_semaphore()
        pl.semaphore_signal(
            barrier_sem, inc=1,
            device_id=partner, device_id_type=pl.DeviceIdType.MESH,
        )

        off_send = (1 - my_x) * n
        off_keep = my_x * n

        cp_sends = []
        for c in range(n_chunks):
            rows = pl.ds(c * mc, mc)
            cp = pltpu.make_async_copy(
                x_ref.at[0, rows, pl.ds(off_send, n)],
                send_buf.at[rows, :],
                local_sems.at[c],
            )
            cp.start()
            cp_sends.append(cp)
        cp_keep = pltpu.make_async_copy(
            x_ref.at[0, :, pl.ds(off_keep, n)], keep_buf, local_sems.at[n_chunks]
        )
        cp_keep.start()

        rdmas = []
        for c, cp in enumerate(cp_sends):
            rows = pl.ds(c * mc, mc)
            cp.wait()
            rdma = pltpu.make_async_remote_copy(
                src_ref=send_buf.at[rows, :],
                dst_ref=recv_buf.at[rows, :],
                send_sem=send_sems.at[c],
                recv_sem=recv_sems.at[c],
                device_id=partner,
                device_id_type=pl.DeviceIdType.MESH,
            )
            rdma.start()
            rdmas.append(rdma)

        pl.semaphore_wait(barrier_sem, 1)
        cp_keep.wait()

        for c, rdma in enumerate(rdmas):
            rows = pl.ds(c * mc, mc)
            rdma.wait_recv()
            out_ref[rows, :] = keep_buf[rows, :] + recv_buf[rows, :]

        for rdma in rdmas:
            rdma.wait_send()

    x = pltpu.with_memory_space_constraint(x, pltpu.MemorySpace.HBM)

    return pl.pallas_call(
        body,
        out_shape=jax.ShapeDtypeStruct((m, n), jnp.float32),
        in_specs=[pl.BlockSpec(memory_space=pl.ANY)],
        out_specs=pl.BlockSpec(memory_space=pltpu.VMEM),
        scratch_shapes=[
            pltpu.VMEM((m, n), jnp.float32),
            pltpu.VMEM((m, n), jnp.float32),
            pltpu.VMEM((m, n), jnp.float32),
            pltpu.SemaphoreType.DMA((3,)),
            pltpu.SemaphoreType.DMA((n_chunks,)),
            pltpu.SemaphoreType.DMA((n_chunks,)),
        ],
        compiler_params=pltpu.CompilerParams(collective_id=0),
    )(x)


# device time: 8394 ns/iter; 1.0348x vs baseline; 1.0060x over previous
import jax
import jax.numpy as jnp
from jax import lax
from jax.experimental import pallas as pl
from jax.experimental.pallas import tpu as pltpu


def kernel(x):
    _, m, n2 = x.shape
    n = n2 // 2

    n_chunks = 4
    mc = m // n_chunks

    def body(x_ref, out_ref, send_buf, keep_buf, recv_buf,
             local_sems, send_sems, recv_sems):
        my_x = lax.axis_index("x")
        my_y = lax.axis_index("y")
        my_z = lax.axis_index("z")
        partner = (1 - my_x, my_y, my_z)

        barrier_sem = pltpu.get_barrier_semaphore()
        pl.semaphore_signal(
            barrier_sem, inc=1,
            device_id=partner, device_id_type=pl.DeviceIdType.MESH,
        )

        off_send = (1 - my_x) * n
        off_keep = my_x * n

        cp_sends = []
        for c in range(n_chunks):
            rows = pl.ds(c * mc, mc)
            cp = pltpu.make_async_copy(
                x_ref.at[0, rows, pl.ds(off_send, n)],
                send_buf.at[rows, :],
                local_sems.at[c],
            )
            cp.start()
            cp_sends.append(cp)
        cp_keep = pltpu.make_async_copy(
            x_ref.at[0, :, pl.ds(off_keep, n)], keep_buf, local_sems.at[n_chunks]
        )
        cp_keep.start()

        rdmas = []
        for c, cp in enumerate(cp_sends):
            rows = pl.ds(c * mc, mc)
            cp.wait()
            rdma = pltpu.make_async_remote_copy(
                src_ref=send_buf.at[rows, :],
                dst_ref=recv_buf.at[rows, :],
                send_sem=send_sems.at[c],
                recv_sem=recv_sems.at[c],
                device_id=partner,
                device_id_type=pl.DeviceIdType.MESH,
            )
            rdma.start()
            rdmas.append(rdma)

        pl.semaphore_wait(barrier_sem, 1)
        cp_keep.wait()

        for c, rdma in enumerate(rdmas):
            rows = pl.ds(c * mc, mc)
            rdma.wait_recv()
            out_ref[rows, :] = keep_buf[rows, :] + recv_buf[rows, :]

        for rdma in rdmas:
            rdma.wait_send()

    x = pltpu.with_memory_space_constraint(x, pltpu.MemorySpace.HBM)

    return pl.pallas_call(
        body,
        out_shape=jax.ShapeDtypeStruct((m, n), jnp.float32),
        in_specs=[pl.BlockSpec(memory_space=pl.ANY)],
        out_specs=pl.BlockSpec(memory_space=pltpu.VMEM),
        scratch_shapes=[
            pltpu.VMEM((m, n), jnp.float32),
            pltpu.VMEM((m, n), jnp.float32),
            pltpu.VMEM((m, n), jnp.float32),
            pltpu.SemaphoreType.DMA((3,)),
            pltpu.SemaphoreType.DMA((n_chunks,)),
            pltpu.SemaphoreType.DMA((n_chunks,)),
        ],
        compiler_params=pltpu.CompilerParams(collective_id=0),
    )(x)


# device time: 7765 ns/iter; 1.1186x vs baseline; 1.0810x over previous
import jax
import jax.numpy as jnp
from jax import lax
from jax.experimental import pallas as pl
from jax.experimental.pallas import tpu as pltpu


def kernel(x):
    _, m, n2 = x.shape
    n = n2 // 2

    n_chunks = 4
    mc = m // n_chunks

    def body(x_ref, out_ref, keep_buf, recv_buf,
             local_sems, send_sems, recv_sems):
        my_x = lax.axis_index("x")
        my_y = lax.axis_index("y")
        my_z = lax.axis_index("z")
        partner = (1 - my_x, my_y, my_z)

        barrier_sem = pltpu.get_barrier_semaphore()
        pl.semaphore_signal(
            barrier_sem, inc=1,
            device_id=partner, device_id_type=pl.DeviceIdType.MESH,
        )

        off_send = (1 - my_x) * n
        off_keep = my_x * n

        rdmas = []
        for c in range(n_chunks):
            rows = pl.ds(c * mc, mc)
            rdma = pltpu.make_async_remote_copy(
                src_ref=x_ref.at[0, rows, pl.ds(off_send, n)],
                dst_ref=recv_buf.at[rows, :],
                send_sem=send_sems.at[c],
                recv_sem=recv_sems.at[c],
                device_id=partner,
                device_id_type=pl.DeviceIdType.MESH,
            )
            rdma.start()
            rdmas.append(rdma)

        cp_keep = pltpu.make_async_copy(
            x_ref.at[0, :, pl.ds(off_keep, n)], keep_buf, local_sems.at[0]
        )
        cp_keep.start()

        pl.semaphore_wait(barrier_sem, 1)
        cp_keep.wait()

        for c, rdma in enumerate(rdmas):
            rows = pl.ds(c * mc, mc)
            rdma.wait_recv()
            out_ref[rows, :] = keep_buf[rows, :] + recv_buf[rows, :]

        for rdma in rdmas:
            rdma.wait_send()

    x = pltpu.with_memory_space_constraint(x, pltpu.MemorySpace.HBM)

    return pl.pallas_call(
        body,
        out_shape=jax.ShapeDtypeStruct((m, n), jnp.float32),
        in_specs=[pl.BlockSpec(memory_space=pl.ANY)],
        out_specs=pl.BlockSpec(memory_space=pltpu.VMEM),
        scratch_shapes=[
            pltpu.VMEM((m, n), jnp.float32),
            pltpu.VMEM((m, n), jnp.float32),
            pltpu.SemaphoreType.DMA((1,)),
            pltpu.SemaphoreType.DMA((n_chunks,)),
            pltpu.SemaphoreType.DMA((n_chunks,)),
        ],
        compiler_params=pltpu.CompilerParams(collective_id=0),
    )(x)


# device time: 7735 ns/iter; 1.1229x vs baseline; 1.0039x over previous
import jax
import jax.numpy as jnp
from jax import lax
from jax.experimental import pallas as pl
from jax.experimental.pallas import tpu as pltpu


def kernel(x):
    _, m, n2 = x.shape
    n = n2 // 2

    n_chunks = 2
    mc = m // n_chunks

    def body(x_ref, out_ref, keep_buf, recv_buf,
             local_sems, send_sems, recv_sems):
        my_x = lax.axis_index("x")
        my_y = lax.axis_index("y")
        my_z = lax.axis_index("z")
        partner = (1 - my_x, my_y, my_z)

        barrier_sem = pltpu.get_barrier_semaphore()
        pl.semaphore_signal(
            barrier_sem, inc=1,
            device_id=partner, device_id_type=pl.DeviceIdType.MESH,
        )

        off_send = (1 - my_x) * n
        off_keep = my_x * n

        rdmas = []
        for c in range(n_chunks):
            rows = pl.ds(c * mc, mc)
            rdma = pltpu.make_async_remote_copy(
                src_ref=x_ref.at[0, rows, pl.ds(off_send, n)],
                dst_ref=recv_buf.at[rows, :],
                send_sem=send_sems.at[c],
                recv_sem=recv_sems.at[c],
                device_id=partner,
                device_id_type=pl.DeviceIdType.MESH,
            )
            rdma.start()
            rdmas.append(rdma)

        cp_keep = pltpu.make_async_copy(
            x_ref.at[0, :, pl.ds(off_keep, n)], keep_buf, local_sems.at[0]
        )
        cp_keep.start()

        pl.semaphore_wait(barrier_sem, 1)
        cp_keep.wait()

        for c, rdma in enumerate(rdmas):
            rows = pl.ds(c * mc, mc)
            rdma.wait_recv()
            out_ref[rows, :] = keep_buf[rows, :] + recv_buf[rows, :]

        for rdma in rdmas:
            rdma.wait_send()

    x = pltpu.with_memory_space_constraint(x, pltpu.MemorySpace.HBM)

    return pl.pallas_call(
        body,
        out_shape=jax.ShapeDtypeStruct((m, n), jnp.float32),
        in_specs=[pl.BlockSpec(memory_space=pl.ANY)],
        out_specs=pl.BlockSpec(memory_space=pltpu.VMEM),
        scratch_shapes=[
            pltpu.VMEM((m, n), jnp.float32),
            pltpu.VMEM((m, n), jnp.float32),
            pltpu.SemaphoreType.DMA((1,)),
            pltpu.SemaphoreType.DMA((n_chunks,)),
            pltpu.SemaphoreType.DMA((n_chunks,)),
        ],
        compiler_params=pltpu.CompilerParams(collective_id=0),
    )(x)


# device time: 7731 ns/iter; 1.1235x vs baseline; 1.0005x over previous
import jax
import jax.numpy as jnp
from jax import lax
from jax.experimental import pallas as pl
from jax.experimental.pallas import tpu as pltpu


def kernel(x):
    _, m, n2 = x.shape
    n = n2 // 2

    n_chunks = 1
    mc = m // n_chunks

    def body(x_ref, out_ref, keep_buf, recv_buf,
             local_sems, send_sems, recv_sems):
        my_x = lax.axis_index("x")
        my_y = lax.axis_index("y")
        my_z = lax.axis_index("z")
        partner = (1 - my_x, my_y, my_z)

        barrier_sem = pltpu.get_barrier_semaphore()
        pl.semaphore_signal(
            barrier_sem, inc=1,
            device_id=partner, device_id_type=pl.DeviceIdType.MESH,
        )

        off_send = (1 - my_x) * n
        off_keep = my_x * n

        rdmas = []
        for c in range(n_chunks):
            rows = pl.ds(c * mc, mc)
            rdma = pltpu.make_async_remote_copy(
                src_ref=x_ref.at[0, rows, pl.ds(off_send, n)],
                dst_ref=recv_buf.at[rows, :],
                send_sem=send_sems.at[c],
                recv_sem=recv_sems.at[c],
                device_id=partner,
                device_id_type=pl.DeviceIdType.MESH,
            )
            rdma.start()
            rdmas.append(rdma)

        cp_keep = pltpu.make_async_copy(
            x_ref.at[0, :, pl.ds(off_keep, n)], keep_buf, local_sems.at[0]
        )
        cp_keep.start()

        pl.semaphore_wait(barrier_sem, 1)
        cp_keep.wait()

        for c, rdma in enumerate(rdmas):
            rows = pl.ds(c * mc, mc)
            rdma.wait_recv()
            out_ref[rows, :] = keep_buf[rows, :] + recv_buf[rows, :]

        for rdma in rdmas:
            rdma.wait_send()

    x = pltpu.with_memory_space_constraint(x, pltpu.MemorySpace.HBM)

    return pl.pallas_call(
        body,
        out_shape=jax.ShapeDtypeStruct((m, n), jnp.float32),
        in_specs=[pl.BlockSpec(memory_space=pl.ANY)],
        out_specs=pl.BlockSpec(memory_space=pltpu.VMEM),
        scratch_shapes=[
            pltpu.VMEM((m, n), jnp.float32),
            pltpu.VMEM((m, n), jnp.float32),
            pltpu.SemaphoreType.DMA((1,)),
            pltpu.SemaphoreType.DMA((n_chunks,)),
            pltpu.SemaphoreType.DMA((n_chunks,)),
        ],
        compiler_params=pltpu.CompilerParams(collective_id=0),
    )(x)
